# Initial kernel scaffold; baseline (speedup 1.0000x reference)
import functools

import jax
import jax.numpy as jnp
from jax import lax
from jax.experimental import pallas as pl
from jax.experimental.pallas import tpu as pltpu
from jax.experimental.pallas import tpu_sc as plsc

N = 10000
E = 320000
D = 128
R = 8
B = 4

NC = 2
NS = 16
NW = NC * NS
CHUNK = 128
CPT = 79
EPT = CPT * CHUNK
E_PAD = EPT * NW
ROWS_PT = N // NS
BN = 400
NB = N // BN


def _xw_body(m_ref, wf_ref, h_ref, out_ref, w_vmem):
    nb = pl.program_id(1)

    @pl.when(nb == 0)
    def _():
        w_vmem[...] = jnp.dot(m_ref[...], wf_ref[...],
                              preferred_element_type=jnp.float32)

    out_ref[0] = jnp.dot(h_ref[...], w_vmem[...],
                         preferred_element_type=jnp.float32)


def _compute_table(h, m_sel, w_flat):
    xw = pl.pallas_call(
        _xw_body,
        grid=(R, NB),
        in_specs=[
            pl.BlockSpec((D, B * D), lambda r, nb: (r, 0)),
            pl.BlockSpec((B * D, D), lambda r, nb: (0, 0)),
            pl.BlockSpec((BN, D), lambda r, nb: (nb, 0)),
        ],
        out_specs=pl.BlockSpec((1, BN, D), lambda r, nb: (r, nb, 0)),
        out_shape=jax.ShapeDtypeStruct((R, N, D), jnp.float32),
        scratch_shapes=[pltpu.VMEM((D, D), jnp.float32)],
    )(m_sel, w_flat, h)
    return xw.reshape(R * N, D)


def _edge_sweep_body(table, gidx_hbm, dst_hbm, norm_hbm, out_hbm,
                     gidx_v, dst_v, norm_v, rows_v, zrow_v, acc, sem):
    cid = lax.axis_index("c")
    sid = lax.axis_index("s")
    wid = sid * NC + cid

    zv = jnp.zeros((16,), jnp.float32)
    for i in range(25):
        for f in range(8):
            zrow_v[i, pl.ds(f * 16, 16)] = zv
    for k in range(ROWS_PT // 25):
        pltpu.sync_copy(zrow_v, acc.at[pl.ds(sid * ROWS_PT + k * 25, 25)])
    plsc.subcore_barrier()

    base = wid * CPT
    pltpu.sync_copy(gidx_hbm.at[pl.ds(base, CPT)], gidx_v)
    pltpu.sync_copy(dst_hbm.at[pl.ds(base, CPT)], dst_v)
    pltpu.sync_copy(norm_hbm.at[pl.ds(base, CPT)], norm_v)

    def chunk_body(j, carry):
        pltpu.async_copy(table.at[gidx_v.at[j]], rows_v, sem).wait()

        def edge_body(e, c2):
            nv = plsc.load_gather(
                norm_v,
                [jnp.full((16,), j, jnp.int32), jnp.full((16,), e, jnp.int32)],
            )
            for f in range(8):
                sl = pl.ds(f * 16, 16)
                rows_v[e, sl] = rows_v[e, sl] * nv
            return c2

        lax.fori_loop(0, CHUNK, edge_body, 0)
        pltpu.sync_copy(rows_v, acc.at[dst_v.at[j]], add=True)
        return carry

    lax.fori_loop(0, CPT, chunk_body, 0)
    plsc.subcore_barrier()

    pltpu.sync_copy(acc.at[pl.ds(sid * ROWS_PT, ROWS_PT)],
                    out_hbm.at[cid, pl.ds(sid * ROWS_PT, ROWS_PT)])


def _edge_sweep(table, gidx2d, dst2d, norm2d):
    mesh = plsc.VectorSubcoreMesh(core_axis_name="c", subcore_axis_name="s")
    return pl.kernel(
        _edge_sweep_body,
        out_type=jax.ShapeDtypeStruct((NC, N, D), jnp.float32),
        mesh=mesh,
        scratch_types=[
            pltpu.VMEM((CPT, CHUNK), jnp.int32),
            pltpu.VMEM((CPT, CHUNK), jnp.int32),
            pltpu.VMEM((CPT, CHUNK), jnp.float32),
            pltpu.VMEM((CHUNK, D), jnp.float32),
            pltpu.VMEM((25, D), jnp.float32),
            pltpu.VMEM_SHARED((N, D), jnp.float32),
            pltpu.SemaphoreType.DMA,
        ],
    )(table, gidx2d, dst2d, norm2d)


def _add_body(p_ref, o_ref):
    o_ref[...] = p_ref[0] + p_ref[1]


def _sum_partials(partials):
    return pl.pallas_call(
        _add_body,
        grid=(NB,),
        in_specs=[pl.BlockSpec((NC, BN, D), lambda nb: (0, nb, 0))],
        out_specs=pl.BlockSpec((BN, D), lambda nb: (nb, 0)),
        out_shape=jax.ShapeDtypeStruct((N, D), jnp.float32),
    )(partials)


def kernel(h, edge_index, rel_type, norm, weight, w_comp):
    s = jnp.arange(R * D)
    cols = (B * (s // R))[:, None] + jnp.arange(B)[None, :]
    vals = jnp.tile(w_comp, (D, 1))
    m_sel = jnp.zeros((R * D, B * D), jnp.float32).at[s[:, None], cols].set(vals)
    w_flat = weight.reshape(B * D, D)

    table = _compute_table(h, m_sel, w_flat)

    src = edge_index[0]
    dst = edge_index[1]
    gidx = rel_type * N + src
    pad = E_PAD - E
    gidx2d = jnp.concatenate([gidx, jnp.zeros((pad,), jnp.int32)]
                             ).reshape(NW * CPT, CHUNK)
    dst2d = jnp.concatenate([dst, jnp.zeros((pad,), jnp.int32)]
                            ).reshape(NW * CPT, CHUNK)
    norm2d = jnp.concatenate([norm[:, 0], jnp.zeros((pad,), jnp.float32)]
                             ).reshape(NW * CPT, CHUNK)

    partials = _edge_sweep(table, gidx2d, dst2d, norm2d)
    return _sum_partials(partials)


# speedup vs baseline: 6.9281x; 6.9281x over previous
import jax
import jax.numpy as jnp
from jax import lax
from jax.experimental import pallas as pl
from jax.experimental.pallas import tpu as pltpu
from jax.experimental.pallas import tpu_sc as plsc

N = 10000
E = 320000
D = 128
R = 8
B = 4

NC = 2
NS = 16
DH = D // NC
CHUNK = 128
CPT = 160
EPT = CPT * CHUNK
E_PAD = EPT * NS
N_PAD = 10240
ROWS_PT = N_PAD // NS
BN = 400
NB = N // BN


def _xw_body(m_ref, wf_ref, h_ref, out_ref, w_vmem):
    nb = pl.program_id(1)

    @pl.when(nb == 0)
    def _():
        w_vmem[...] = jnp.dot(m_ref[...], wf_ref[...],
                              preferred_element_type=jnp.float32)

    res = jnp.dot(h_ref[...], w_vmem[...], preferred_element_type=jnp.float32)
    out_ref[0, 0] = res[:, :DH]
    out_ref[0, 1] = res[:, DH:]


def _compute_table(h, m_sel, w_flat):
    xw = pl.pallas_call(
        _xw_body,
        grid=(R, NB),
        in_specs=[
            pl.BlockSpec((D, B * D), lambda r, nb: (r, 0)),
            pl.BlockSpec((B * D, D), lambda r, nb: (0, 0)),
            pl.BlockSpec((BN, D), lambda r, nb: (nb, 0)),
        ],
        out_specs=pl.BlockSpec((1, NC, BN, DH), lambda r, nb: (r, 0, nb, 0)),
        out_shape=jax.ShapeDtypeStruct((R, NC, N, DH), jnp.float32),
        scratch_shapes=[pltpu.VMEM((D, D), jnp.float32)],
    )(m_sel, w_flat, h)
    return xw.reshape(R * NC * N, DH)


def _edge_sweep_body(table, gidx_hbm, dst_hbm, norm_hbm, out_hbm,
                     gidx_v, dst_v, norm_v, rows_v, zrow_v, acc, sem):
    cid = lax.axis_index("c")
    sid = lax.axis_index("s")

    zv = jnp.zeros((16,), jnp.float32)
    for i in range(32):
        for f in range(DH // 16):
            zrow_v[i, pl.ds(f * 16, 16)] = zv
    for k in range(ROWS_PT // 32):
        pltpu.sync_copy(zrow_v, acc.at[pl.ds(sid * ROWS_PT + k * 32, 32)])
    plsc.subcore_barrier()

    base = sid * CPT
    pltpu.sync_copy(gidx_hbm.at[cid, pl.ds(base, CPT)], gidx_v)
    pltpu.sync_copy(dst_hbm.at[pl.ds(base, CPT)], dst_v)
    pltpu.sync_copy(norm_hbm.at[pl.ds(sid * EPT, EPT)], norm_v)

    dnums = lax.GatherDimensionNumbers(
        offset_dims=(), collapsed_slice_dims=(0,), start_index_map=(0,))

    def chunk_body(j, carry):
        pltpu.async_copy(table.at[gidx_v.at[j]], rows_v, sem).wait()

        def group_body(g, c2):
            kv = norm_v[pl.ds(j * CHUNK + g * 16, 16)]
            for lane in range(16):
                nv = lax.gather(kv, jnp.full((16, 1), lane, jnp.int32),
                                dnums, (1,),
                                mode=lax.GatherScatterMode.PROMISE_IN_BOUNDS)
                e = g * 16 + lane
                for f in range(DH // 16):
                    sl = pl.ds(f * 16, 16)
                    rows_v[e, sl] = rows_v[e, sl] * nv
            return c2

        lax.fori_loop(0, CHUNK // 16, group_body, 0)
        pltpu.sync_copy(rows_v, acc.at[dst_v.at[j]], add=True)
        return carry

    lax.fori_loop(0, CPT, chunk_body, 0)
    plsc.subcore_barrier()

    pltpu.sync_copy(acc.at[pl.ds(sid * ROWS_PT, ROWS_PT)],
                    out_hbm.at[cid, pl.ds(sid * ROWS_PT, ROWS_PT)])


def _edge_sweep(table, gidx_pair, dst2d, norm1d):
    mesh = plsc.VectorSubcoreMesh(core_axis_name="c", subcore_axis_name="s")
    return pl.kernel(
        _edge_sweep_body,
        out_type=jax.ShapeDtypeStruct((NC, N_PAD, DH), jnp.float32),
        mesh=mesh,
        compiler_params=pltpu.CompilerParams(use_tc_tiling_on_sc=False),
        scratch_types=[
            pltpu.VMEM((CPT, CHUNK), jnp.int32),
            pltpu.VMEM((CPT, CHUNK), jnp.int32),
            pltpu.VMEM((EPT,), jnp.float32),
            pltpu.VMEM((CHUNK, DH), jnp.float32),
            pltpu.VMEM((32, DH), jnp.float32),
            pltpu.VMEM_SHARED((N_PAD, DH), jnp.float32),
            pltpu.SemaphoreType.DMA,
        ],
    )(table, gidx_pair, dst2d, norm1d)


def kernel(h, edge_index, rel_type, norm, weight, w_comp):
    s = jnp.arange(R * D)
    cols = (B * (s // R))[:, None] + jnp.arange(B)[None, :]
    vals = jnp.tile(w_comp, (D, 1))
    m_sel = jnp.zeros((R * D, B * D), jnp.float32).at[s[:, None], cols].set(vals)
    w_flat = weight.reshape(B * D, D)

    table = _compute_table(h, m_sel, w_flat)

    src = edge_index[0]
    dst = edge_index[1]
    pad = E_PAD - E
    gidx = rel_type * (NC * N) + src
    gidx_p = jnp.concatenate([gidx, jnp.zeros((pad,), jnp.int32)])
    gidx_pair = (gidx_p[None, :] +
                 (jnp.arange(NC, dtype=jnp.int32) * N)[:, None]
                 ).reshape(NC, NS * CPT, CHUNK)
    dst2d = jnp.concatenate([dst, jnp.zeros((pad,), jnp.int32)]
                            ).reshape(NS * CPT, CHUNK)
    norm1d = jnp.concatenate([norm[:, 0], jnp.zeros((pad,), jnp.float32)])

    halves = _edge_sweep(table, gidx_pair, dst2d, norm1d)
    return jnp.concatenate([halves[0, :N], halves[1, :N]], axis=1)


# speedup vs baseline: 9.8371x; 1.4199x over previous
import jax
import jax.numpy as jnp
from jax import lax
from jax.experimental import pallas as pl
from jax.experimental.pallas import tpu as pltpu
from jax.experimental.pallas import tpu_sc as plsc

N = 10000
E = 320000
D = 128
R = 8
B = 4

NC = 2
NS = 16
DH = D // NC
CHUNK = 128
CPT = 160
EPT = CPT * CHUNK
E_PAD = EPT * NS
N_PAD = 10240
ROWS_PT = N_PAD // NS
BN = 400
NB = N // BN


def _xw_body(m_ref, wf_ref, h_ref, out_ref, w_vmem):
    nb = pl.program_id(1)

    @pl.when(nb == 0)
    def _():
        w_vmem[...] = jnp.dot(m_ref[...], wf_ref[...],
                              preferred_element_type=jnp.float32)

    res = jnp.dot(h_ref[...], w_vmem[...], preferred_element_type=jnp.float32)
    out_ref[0, 0] = res[:, :DH]
    out_ref[0, 1] = res[:, DH:]


def _compute_table(h, m_sel, w_flat):
    xw = pl.pallas_call(
        _xw_body,
        grid=(R, NB),
        in_specs=[
            pl.BlockSpec((D, B * D), lambda r, nb: (r, 0)),
            pl.BlockSpec((B * D, D), lambda r, nb: (0, 0)),
            pl.BlockSpec((BN, D), lambda r, nb: (nb, 0)),
        ],
        out_specs=pl.BlockSpec((1, NC, BN, DH), lambda r, nb: (r, 0, nb, 0)),
        out_shape=jax.ShapeDtypeStruct((R, NC, N, DH), jnp.float32),
        scratch_shapes=[pltpu.VMEM((D, D), jnp.float32)],
    )(m_sel, w_flat, h)
    return xw.reshape(R * NC * N, DH)


NBUF = 4
PCH = 40


def _edge_sweep_body(table, gidx_hbm, dst_hbm, norm_hbm, out_hbm,
                     gidx_v, dst_v, norm_v, rows_v, zrow_v, acc,
                     gsem, ssem):
    cid = lax.axis_index("c")
    sid = lax.axis_index("s")

    zv = jnp.zeros((16,), jnp.float32)
    for i in range(32):
        for f in range(DH // 16):
            zrow_v[i, pl.ds(f * 16, 16)] = zv
    for k in range(ROWS_PT // 32):
        pltpu.sync_copy(zrow_v, acc.at[pl.ds(sid * ROWS_PT + k * 32, 32)])
    plsc.subcore_barrier()

    dnums = lax.GatherDimensionNumbers(
        offset_dims=(), collapsed_slice_dims=(0,), start_index_map=(0,))

    def start_gather(j, b):
        return pltpu.async_copy(table.at[gidx_v.at[j]], rows_v.at[b],
                                gsem.at[b])

    def wait_gather(b):
        pltpu.make_async_copy(table.at[gidx_v.at[0]], rows_v.at[b],
                              gsem.at[b]).wait()

    def start_scatter(j, b):
        return pltpu.async_copy(rows_v.at[b], acc.at[dst_v.at[j]],
                                ssem.at[b], add=True)

    def wait_scatter(b):
        pltpu.make_async_copy(rows_v.at[b], acc.at[dst_v.at[0]],
                              ssem.at[b]).wait()

    def scale(j, b):
        def group_body(g, c2):
            kv = norm_v[pl.ds(j * CHUNK + g * 16, 16)]
            for lane in range(16):
                nv = lax.gather(kv, jnp.full((16, 1), lane, jnp.int32),
                                dnums, (1,),
                                mode=lax.GatherScatterMode.PROMISE_IN_BOUNDS)
                e = g * 16 + lane
                for f in range(DH // 16):
                    sl = pl.ds(f * 16, 16)
                    rows_v[b, e, sl] = rows_v[b, e, sl] * nv
            return c2

        lax.fori_loop(0, CHUNK // 16, group_body, 0)

    def run_pass(p, carry):
        base = sid * CPT + p * PCH
        pltpu.sync_copy(gidx_hbm.at[cid, pl.ds(base, PCH)], gidx_v)
        pltpu.sync_copy(dst_hbm.at[pl.ds(base, PCH)], dst_v)
        pltpu.sync_copy(norm_hbm.at[pl.ds(base * CHUNK, PCH * CHUNK)], norm_v)

        start_gather(0, 0)
        start_gather(1, 1)
        for j in (0, 1):
            wait_gather(j)
            start_gather(j + 2, j + 2)
            scale(j, j)
            start_scatter(j, j)

        def steady(j, c2):
            b = j % NBUF
            bn = (j + 2) % NBUF
            wait_gather(b)
            wait_scatter(bn)
            start_gather(j + 2, bn)
            scale(j, b)
            start_scatter(j, b)
            return c2

        lax.fori_loop(2, PCH - 2, steady, 0)

        for j in (PCH - 2, PCH - 1):
            b = j % NBUF
            wait_gather(b)
            wait_scatter((j + 2) % NBUF)
            scale(j, b)
            start_scatter(j, b)
        wait_scatter((PCH - 2) % NBUF)
        wait_scatter((PCH - 1) % NBUF)
        return carry

    lax.fori_loop(0, CPT // PCH, run_pass, 0)
    plsc.subcore_barrier()

    pltpu.sync_copy(acc.at[pl.ds(sid * ROWS_PT, ROWS_PT)],
                    out_hbm.at[cid, pl.ds(sid * ROWS_PT, ROWS_PT)])


def _edge_sweep(table, gidx_pair, dst2d, norm1d):
    mesh = plsc.VectorSubcoreMesh(core_axis_name="c", subcore_axis_name="s")
    return pl.kernel(
        _edge_sweep_body,
        out_type=jax.ShapeDtypeStruct((NC, N_PAD, DH), jnp.float32),
        mesh=mesh,
        compiler_params=pltpu.CompilerParams(use_tc_tiling_on_sc=False),
        scratch_types=[
            pltpu.VMEM((PCH, CHUNK), jnp.int32),
            pltpu.VMEM((PCH, CHUNK), jnp.int32),
            pltpu.VMEM((PCH * CHUNK,), jnp.float32),
            pltpu.VMEM((NBUF, CHUNK, DH), jnp.float32),
            pltpu.VMEM((32, DH), jnp.float32),
            pltpu.VMEM_SHARED((N_PAD, DH), jnp.float32),
            pltpu.SemaphoreType.DMA((NBUF,)),
            pltpu.SemaphoreType.DMA((NBUF,)),
        ],
    )(table, gidx_pair, dst2d, norm1d)


def kernel(h, edge_index, rel_type, norm, weight, w_comp):
    s = jnp.arange(R * D)
    cols = (B * (s // R))[:, None] + jnp.arange(B)[None, :]
    vals = jnp.tile(w_comp, (D, 1))
    m_sel = jnp.zeros((R * D, B * D), jnp.float32).at[s[:, None], cols].set(vals)
    w_flat = weight.reshape(B * D, D)

    table = _compute_table(h, m_sel, w_flat)

    src = edge_index[0]
    dst = edge_index[1]
    pad = E_PAD - E
    gidx = rel_type * (NC * N) + src
    gidx_p = jnp.concatenate([gidx, jnp.zeros((pad,), jnp.int32)])
    gidx_pair = (gidx_p[None, :] +
                 (jnp.arange(NC, dtype=jnp.int32) * N)[:, None]
                 ).reshape(NC, NS * CPT, CHUNK)
    dst2d = jnp.concatenate([dst, jnp.zeros((pad,), jnp.int32)]
                            ).reshape(NS * CPT, CHUNK)
    norm1d = jnp.concatenate([norm[:, 0], jnp.zeros((pad,), jnp.float32)])

    halves = _edge_sweep(table, gidx_pair, dst2d, norm1d)
    return jnp.concatenate([halves[0, :N], halves[1, :N]], axis=1)


# speedup vs baseline: 11.9375x; 1.2135x over previous
import jax
import jax.numpy as jnp
from jax import lax
from jax.experimental import pallas as pl
from jax.experimental.pallas import tpu as pltpu
from jax.experimental.pallas import tpu_sc as plsc

N = 10000
E = 320000
D = 128
R = 8
B = 4

NC = 2
NS = 16
DH = D // NC
CHUNK = 128
CPT = 160
EPT = CPT * CHUNK
E_PAD = EPT * NS
N_PAD = 10240
ROWS_PT = N_PAD // NS
BN = 400
NB = N // BN


def _xw_body(m_ref, wf_ref, h_ref, out_ref):
    w_r = jnp.dot(m_ref[...], wf_ref[...], preferred_element_type=jnp.float32)
    res = jnp.dot(h_ref[...], w_r, preferred_element_type=jnp.float32)
    out_ref[0, 0] = res[:, :DH]
    out_ref[0, 1] = res[:, DH:]


def _compute_table(h, m_sel, w_flat):
    xw = pl.pallas_call(
        _xw_body,
        grid=(R,),
        in_specs=[
            pl.BlockSpec((D, B * D), lambda r: (r, 0)),
            pl.BlockSpec((B * D, D), lambda r: (0, 0)),
            pl.BlockSpec((N, D), lambda r: (0, 0)),
        ],
        out_specs=pl.BlockSpec((1, NC, N, DH), lambda r: (r, 0, 0, 0)),
        out_shape=jax.ShapeDtypeStruct((R, NC, N, DH), jnp.float32),
    )(m_sel, w_flat, h)
    return xw.reshape(R * NC * N, DH)


NBUF = 4
PCH = 40


def _edge_sweep_body(table, gidx_hbm, dst_hbm, norm_hbm, out_hbm,
                     gidx_v, dst_v, norm_v, rows_v, zrow_v, acc,
                     gsem, ssem):
    cid = lax.axis_index("c")
    sid = lax.axis_index("s")

    zv = jnp.zeros((16,), jnp.float32)
    for i in range(32):
        for f in range(DH // 16):
            zrow_v[i, pl.ds(f * 16, 16)] = zv
    for k in range(ROWS_PT // 32):
        pltpu.sync_copy(zrow_v, acc.at[pl.ds(sid * ROWS_PT + k * 32, 32)])
    plsc.subcore_barrier()

    dnums = lax.GatherDimensionNumbers(
        offset_dims=(), collapsed_slice_dims=(0,), start_index_map=(0,))

    def start_gather(j, b):
        return pltpu.async_copy(table.at[gidx_v.at[j]], rows_v.at[b],
                                gsem.at[b])

    def wait_gather(b):
        pltpu.make_async_copy(table.at[gidx_v.at[0]], rows_v.at[b],
                              gsem.at[b]).wait()

    def start_scatter(j, b):
        return pltpu.async_copy(rows_v.at[b], acc.at[dst_v.at[j]],
                                ssem.at[b], add=True)

    def wait_scatter(b):
        pltpu.make_async_copy(rows_v.at[b], acc.at[dst_v.at[0]],
                              ssem.at[b]).wait()

    def scale(j, b):
        def group_body(g, c2):
            kv = norm_v[pl.ds(j * CHUNK + g * 16, 16)]
            for lane in range(16):
                nv = lax.gather(kv, jnp.full((16, 1), lane, jnp.int32),
                                dnums, (1,),
                                mode=lax.GatherScatterMode.PROMISE_IN_BOUNDS)
                e = g * 16 + lane
                for f in range(DH // 16):
                    sl = pl.ds(f * 16, 16)
                    rows_v[b, e, sl] = rows_v[b, e, sl] * nv
            return c2

        lax.fori_loop(0, CHUNK // 16, group_body, 0)

    def run_pass(p, carry):
        base = sid * CPT + p * PCH
        pltpu.sync_copy(gidx_hbm.at[cid, pl.ds(base, PCH)], gidx_v)
        pltpu.sync_copy(dst_hbm.at[pl.ds(base, PCH)], dst_v)
        pltpu.sync_copy(norm_hbm.at[pl.ds(base * CHUNK, PCH * CHUNK)], norm_v)

        start_gather(0, 0)
        start_gather(1, 1)
        for j in (0, 1):
            wait_gather(j)
            start_gather(j + 2, j + 2)
            scale(j, j)
            start_scatter(j, j)

        def steady(j, c2):
            b = j % NBUF
            bn = (j + 2) % NBUF
            wait_gather(b)
            wait_scatter(bn)
            start_gather(j + 2, bn)
            scale(j, b)
            start_scatter(j, b)
            return c2

        lax.fori_loop(2, PCH - 2, steady, 0)

        for j in (PCH - 2, PCH - 1):
            b = j % NBUF
            wait_gather(b)
            wait_scatter((j + 2) % NBUF)
            scale(j, b)
            start_scatter(j, b)
        wait_scatter((PCH - 2) % NBUF)
        wait_scatter((PCH - 1) % NBUF)
        return carry

    lax.fori_loop(0, CPT // PCH, run_pass, 0)
    plsc.subcore_barrier()

    pltpu.sync_copy(acc.at[pl.ds(sid * ROWS_PT, ROWS_PT)],
                    out_hbm.at[cid, pl.ds(sid * ROWS_PT, ROWS_PT)])


def _edge_sweep(table, gidx_pair, dst2d, norm1d):
    mesh = plsc.VectorSubcoreMesh(core_axis_name="c", subcore_axis_name="s")
    return pl.kernel(
        _edge_sweep_body,
        out_type=jax.ShapeDtypeStruct((NC, N_PAD, DH), jnp.float32),
        mesh=mesh,
        compiler_params=pltpu.CompilerParams(use_tc_tiling_on_sc=False),
        scratch_types=[
            pltpu.VMEM((PCH, CHUNK), jnp.int32),
            pltpu.VMEM((PCH, CHUNK), jnp.int32),
            pltpu.VMEM((PCH * CHUNK,), jnp.float32),
            pltpu.VMEM((NBUF, CHUNK, DH), jnp.float32),
            pltpu.VMEM((32, DH), jnp.float32),
            pltpu.VMEM_SHARED((N_PAD, DH), jnp.float32),
            pltpu.SemaphoreType.DMA((NBUF,)),
            pltpu.SemaphoreType.DMA((NBUF,)),
        ],
    )(table, gidx_pair, dst2d, norm1d)


def kernel(h, edge_index, rel_type, norm, weight, w_comp):
    s = jnp.arange(R * D)
    cols = (B * (s // R))[:, None] + jnp.arange(B)[None, :]
    vals = jnp.tile(w_comp, (D, 1))
    m_sel = jnp.zeros((R * D, B * D), jnp.float32).at[s[:, None], cols].set(vals)
    w_flat = weight.reshape(B * D, D)

    table = _compute_table(h, m_sel, w_flat)

    src = edge_index[0]
    dst = edge_index[1]
    pad = E_PAD - E
    gidx = rel_type * (NC * N) + src
    gidx_p = jnp.concatenate([gidx, jnp.zeros((pad,), jnp.int32)])
    gidx_pair = (gidx_p[None, :] +
                 (jnp.arange(NC, dtype=jnp.int32) * N)[:, None]
                 ).reshape(NC, NS * CPT, CHUNK)
    dst2d = jnp.concatenate([dst, jnp.zeros((pad,), jnp.int32)]
                            ).reshape(NS * CPT, CHUNK)
    norm1d = jnp.concatenate([norm[:, 0], jnp.zeros((pad,), jnp.float32)])

    halves = _edge_sweep(table, gidx_pair, dst2d, norm1d)
    return jnp.concatenate([halves[0, :N], halves[1, :N]], axis=1)


# speedup vs baseline: 14.5053x; 1.2151x over previous
import jax
import jax.numpy as jnp
from jax import lax
from jax.experimental import pallas as pl
from jax.experimental.pallas import tpu as pltpu
from jax.experimental.pallas import tpu_sc as plsc

N = 10000
E = 320000
D = 128
R = 8
B = 4

NC = 2
NS = 16
DH = D // NC
CHUNK = 128
CPT = 160
EPT = CPT * CHUNK
E_PAD = EPT * NS
N_PAD = 10240
ROWS_PT = N_PAD // NS
BN = 400
NB = N // BN


def _xw_body(m_ref, wf_ref, h_ref, out_ref):
    w_r = jnp.dot(m_ref[...], wf_ref[...], preferred_element_type=jnp.float32)
    res = jnp.dot(h_ref[...], w_r, preferred_element_type=jnp.float32)
    out_ref[0, 0] = res[:, :DH].astype(jnp.bfloat16)
    out_ref[0, 1] = res[:, DH:].astype(jnp.bfloat16)


def _compute_table(h, m_sel, w_flat):
    xw = pl.pallas_call(
        _xw_body,
        grid=(R,),
        in_specs=[
            pl.BlockSpec((D, B * D), lambda r: (r, 0)),
            pl.BlockSpec((B * D, D), lambda r: (0, 0)),
            pl.BlockSpec((N, D), lambda r: (0, 0)),
        ],
        out_specs=pl.BlockSpec((1, NC, N, DH), lambda r: (r, 0, 0, 0)),
        out_shape=jax.ShapeDtypeStruct((R, NC, N, DH), jnp.bfloat16),
    )(m_sel, w_flat, h)
    return xw.reshape(R * NC * N, DH)


NBUF = 4
SBUF = 2
PCH = 40


def _edge_sweep_body(table, gidx_hbm, dst_hbm, norm_hbm, out_hbm,
                     gidx_v, dst_v, norm_v, rows_v, stage_v, zrow_v, acc,
                     gsem, ssem):
    cid = lax.axis_index("c")
    sid = lax.axis_index("s")

    zv = jnp.zeros((16,), jnp.float32)
    for i in range(32):
        for f in range(DH // 16):
            zrow_v[i, pl.ds(f * 16, 16)] = zv
    for k in range(ROWS_PT // 32):
        pltpu.sync_copy(zrow_v, acc.at[pl.ds(sid * ROWS_PT + k * 32, 32)])
    plsc.subcore_barrier()

    dnums = lax.GatherDimensionNumbers(
        offset_dims=(), collapsed_slice_dims=(0,), start_index_map=(0,))

    def start_gather(j, b):
        return pltpu.async_copy(table.at[gidx_v.at[j]], rows_v.at[b],
                                gsem.at[b])

    def wait_gather(b):
        pltpu.make_async_copy(table.at[gidx_v.at[0]], rows_v.at[b],
                              gsem.at[b]).wait()

    def start_scatter(j, sb):
        return pltpu.async_copy(stage_v.at[sb], acc.at[dst_v.at[j]],
                                ssem.at[sb], add=True)

    def wait_scatter(sb):
        pltpu.make_async_copy(stage_v.at[sb], acc.at[dst_v.at[0]],
                              ssem.at[sb]).wait()

    def scale(j, b, sb):
        def group_body(g, c2):
            kv = norm_v[pl.ds(j * CHUNK + g * 16, 16)]
            for lane in range(16):
                nv = lax.gather(kv, jnp.full((16, 1), lane, jnp.int32),
                                dnums, (1,),
                                mode=lax.GatherScatterMode.PROMISE_IN_BOUNDS)
                e = g * 16 + lane
                for k in range(DH // 32):
                    c = rows_v[b, e, pl.ds(k * 2, 2), :].astype(jnp.float32)
                    stage_v[sb, e, pl.ds(k * 32, 16)] = c[0] * nv
                    stage_v[sb, e, pl.ds(k * 32 + 16, 16)] = c[1] * nv
            return c2

        lax.fori_loop(0, CHUNK // 16, group_body, 0)

    def run_pass(p, carry):
        base = sid * CPT + p * PCH
        pltpu.sync_copy(gidx_hbm.at[cid, pl.ds(base, PCH)], gidx_v)
        pltpu.sync_copy(dst_hbm.at[pl.ds(base, PCH)], dst_v)
        pltpu.sync_copy(norm_hbm.at[pl.ds(base * CHUNK, PCH * CHUNK)], norm_v)

        start_gather(0, 0)
        start_gather(1, 1)
        for j in (0, 1):
            wait_gather(j)
            start_gather(j + 2, j + 2)
            scale(j, j, j % SBUF)
            start_scatter(j, j % SBUF)

        def steady(j, c2):
            b = j % NBUF
            bn = (j + 2) % NBUF
            sb = j % SBUF
            wait_gather(b)
            wait_scatter(sb)
            start_gather(j + 2, bn)
            scale(j, b, sb)
            start_scatter(j, sb)
            return c2

        lax.fori_loop(2, PCH - 2, steady, 0)

        for j in (PCH - 2, PCH - 1):
            b = j % NBUF
            sb = j % SBUF
            wait_gather(b)
            wait_scatter(sb)
            scale(j, b, sb)
            start_scatter(j, sb)
        wait_scatter(0)
        wait_scatter(1)
        return carry

    lax.fori_loop(0, CPT // PCH, run_pass, 0)
    plsc.subcore_barrier()

    pltpu.sync_copy(acc.at[pl.ds(sid * ROWS_PT, ROWS_PT)],
                    out_hbm.at[cid, pl.ds(sid * ROWS_PT, ROWS_PT)])


def _edge_sweep(table, gidx_pair, dst2d, norm1d):
    mesh = plsc.VectorSubcoreMesh(core_axis_name="c", subcore_axis_name="s")
    return pl.kernel(
        _edge_sweep_body,
        out_type=jax.ShapeDtypeStruct((NC, N_PAD, DH), jnp.float32),
        mesh=mesh,
        compiler_params=pltpu.CompilerParams(use_tc_tiling_on_sc=False),
        scratch_types=[
            pltpu.VMEM((PCH, CHUNK), jnp.int32),
            pltpu.VMEM((PCH, CHUNK), jnp.int32),
            pltpu.VMEM((PCH * CHUNK,), jnp.float32),
            pltpu.VMEM((NBUF, CHUNK, 4, 16), jnp.bfloat16),
            pltpu.VMEM((SBUF, CHUNK, DH), jnp.float32),
            pltpu.VMEM((32, DH), jnp.float32),
            pltpu.VMEM_SHARED((N_PAD, DH), jnp.float32),
            pltpu.SemaphoreType.DMA((NBUF,)),
            pltpu.SemaphoreType.DMA((SBUF,)),
        ],
    )(table, gidx_pair, dst2d, norm1d)


def kernel(h, edge_index, rel_type, norm, weight, w_comp):
    s = jnp.arange(R * D)
    cols = (B * (s // R))[:, None] + jnp.arange(B)[None, :]
    vals = jnp.tile(w_comp, (D, 1))
    m_sel = jnp.zeros((R * D, B * D), jnp.float32).at[s[:, None], cols].set(vals)
    w_flat = weight.reshape(B * D, D)

    table = _compute_table(h, m_sel, w_flat).reshape(R * NC * N, 4, 16)

    src = edge_index[0]
    dst = edge_index[1]
    pad = E_PAD - E
    gidx = rel_type * (NC * N) + src
    gidx_p = jnp.concatenate([gidx, jnp.zeros((pad,), jnp.int32)])
    gidx_pair = (gidx_p[None, :] +
                 (jnp.arange(NC, dtype=jnp.int32) * N)[:, None]
                 ).reshape(NC, NS * CPT, CHUNK)
    dst2d = jnp.concatenate([dst, jnp.zeros((pad,), jnp.int32)]
                            ).reshape(NS * CPT, CHUNK)
    norm1d = jnp.concatenate([norm[:, 0], jnp.zeros((pad,), jnp.float32)])

    halves = _edge_sweep(table, gidx_pair, dst2d, norm1d)
    return jnp.concatenate([halves[0, :N], halves[1, :N]], axis=1)
